# baseline (speedup 1.0000x reference)
import functools

import jax
import jax.numpy as jnp
from jax import lax
from jax.experimental import pallas as pl
from jax.experimental.pallas import tpu as pltpu
from jax.experimental.pallas import tpu_sc as plsc

N_NODES = 10000
N_EDGES = 320000
F = 128
L = 16

NC = 2
NS = 16
NW = NC * NS

CH = 128
EPT = 10240
NCHUNK = EPT // CH
E_PAD = EPT * NW
NCK0 = 136
NCK1 = 2 * NCHUNK - NCK0
NCK_MAX = 160
SHIFT = 14
MASK = (1 << SHIFT) - 1
ACC_ROWS = 10240
ROWS_PER_TILE = ACC_ROWS // NS
DUMMY_DST = N_NODES


def _matmul_body(x_ref, w_ref, o_ref):
    o_ref[...] = jnp.dot(x_ref[...], w_ref[...],
                         preferred_element_type=jnp.float32)


def _combine_body(p0_ref, p1_ref, b_ref, o_ref):
    o_ref[...] = p0_ref[...] + p1_ref[...] + b_ref[...]


def _sc_scatter_kernel(support_hbm, packed_hbm, out_hbm,
                       packed_v, sring_v, dring_v, rows_v, acc_sh, sem):
    c = lax.axis_index("c")
    s = lax.axis_index("s")

    zero16 = jnp.zeros((L,), jnp.float32)

    def _zero_row(i, carry):
        for l in range(F // L):
            rows_v[i, pl.ds(l * L, L)] = zero16
        return carry

    lax.fori_loop(0, CH, _zero_row, 0)
    base = s * ROWS_PER_TILE
    for k in range(ROWS_PER_TILE // CH):
        pltpu.sync_copy(rows_v, acc_sh.at[pl.ds(base + k * CH, CH)])
    plsc.subcore_barrier()

    wid = c * NS + s
    pltpu.sync_copy(packed_hbm.at[wid], packed_v)

    def _chunk(j, carry):
        for l in range(F // L):
            v = packed_v[j, pl.ds(l * L, L)]
            sring_v[0, pl.ds(l * L, L)] = v & MASK
            dring_v[0, pl.ds(l * L, L)] = lax.shift_right_logical(v, SHIFT)
        pltpu.async_copy(support_hbm.at[sring_v.at[0]], rows_v, sem).wait()
        pltpu.sync_copy(rows_v, acc_sh.at[dring_v.at[0]], add=True)
        return carry

    lax.fori_loop(0, jnp.where(c == 0, NCK0, NCK1), _chunk, 0)

    plsc.subcore_barrier()
    pltpu.sync_copy(acc_sh.at[pl.ds(base, ROWS_PER_TILE)],
                    out_hbm.at[c, pl.ds(base, ROWS_PER_TILE)])


_sc_scatter = functools.partial(
    pl.kernel,
    out_type=jax.ShapeDtypeStruct((NC, ACC_ROWS, F), jnp.float32),
    mesh=plsc.VectorSubcoreMesh(core_axis_name="c", subcore_axis_name="s"),
    scratch_types=[
        pltpu.VMEM((NCK_MAX, CH), jnp.int32),
        pltpu.VMEM((2, CH), jnp.int32),
        pltpu.VMEM((2, CH), jnp.int32),
        pltpu.VMEM((CH, F), jnp.float32),
        pltpu.VMEM_SHARED((ACC_ROWS, F), jnp.float32),
        pltpu.SemaphoreType.DMA,
    ],
)(_sc_scatter_kernel)


def kernel(h_v, edge_index, weight, bias):
    rows_blk = 1000
    support = pl.pallas_call(
        _matmul_body,
        grid=(N_NODES // rows_blk,),
        in_specs=[
            pl.BlockSpec((rows_blk, F), lambda i: (i, 0)),
            pl.BlockSpec((F, F), lambda i: (0, 0)),
        ],
        out_specs=pl.BlockSpec((rows_blk, F), lambda i: (i, 0)),
        out_shape=jax.ShapeDtypeStruct((N_NODES, F), jnp.float32),
    )(h_v, weight)

    ei = edge_index.astype(jnp.int32)
    flat = (jnp.pad(ei[0], (0, E_PAD - N_EDGES))
            | (jnp.pad(ei[1], (0, E_PAD - N_EDGES),
                       constant_values=DUMMY_DST) << SHIFT))

    fill = DUMMY_DST << SHIFT
    cut = NS * NCK0 * CH
    p0 = flat[:cut].reshape(NS, NCK0, CH)
    p1 = flat[cut:].reshape(NS, NCK1, CH)
    p0 = jnp.pad(p0, ((0, 0), (0, NCK_MAX - NCK0), (0, 0)),
                 constant_values=fill)
    p1 = jnp.pad(p1, ((0, 0), (0, NCK_MAX - NCK1), (0, 0)),
                 constant_values=fill)
    packed = jnp.concatenate([p0, p1], axis=0)

    partials = _sc_scatter(support, packed)

    out = pl.pallas_call(
        _combine_body,
        grid=(N_NODES // rows_blk,),
        in_specs=[
            pl.BlockSpec((rows_blk, F), lambda i: (i, 0)),
            pl.BlockSpec((rows_blk, F), lambda i: (i, 0)),
            pl.BlockSpec((1, F), lambda i: (0, 0)),
        ],
        out_specs=pl.BlockSpec((rows_blk, F), lambda i: (i, 0)),
        out_shape=jax.ShapeDtypeStruct((N_NODES, F), jnp.float32),
    )(partials[0, :N_NODES], partials[1, :N_NODES], bias.reshape(1, F))
    return out


# speedup vs baseline: 1.0258x; 1.0258x over previous
import functools

import jax
import jax.numpy as jnp
from jax import lax
from jax.experimental import pallas as pl
from jax.experimental.pallas import tpu as pltpu
from jax.experimental.pallas import tpu_sc as plsc

N_NODES = 10000
N_EDGES = 320000
F = 128
L = 16

NC = 2
NS = 16
NW = NC * NS

CH = 128
EPT = 10240
NCHUNK = EPT // CH
E_PAD = EPT * NW
NCK0 = 140
NCK1 = 2 * NCHUNK - NCK0
NCK_MAX = 160
SHIFT = 14
MASK = (1 << SHIFT) - 1
ACC_ROWS = 10240
ROWS_PER_TILE = ACC_ROWS // NS
DUMMY_DST = N_NODES


def _matmul_body(x_ref, w_ref, o_ref):
    o_ref[...] = jnp.dot(x_ref[...], w_ref[...],
                         preferred_element_type=jnp.float32)


def _combine_body(p0_ref, p1_ref, b_ref, o_ref):
    o_ref[...] = p0_ref[...] + p1_ref[...] + b_ref[...]


def _sc_scatter_kernel(support_hbm, packed_hbm, out_hbm,
                       packed_v, sring_v, dring_v, rows_v, acc_sh, sem):
    c = lax.axis_index("c")
    s = lax.axis_index("s")

    zero16 = jnp.zeros((L,), jnp.float32)

    def _zero_row(i, carry):
        for l in range(F // L):
            rows_v[i, pl.ds(l * L, L)] = zero16
        return carry

    lax.fori_loop(0, CH, _zero_row, 0)
    base = s * ROWS_PER_TILE
    for k in range(ROWS_PER_TILE // CH):
        pltpu.sync_copy(rows_v, acc_sh.at[pl.ds(base + k * CH, CH)])
    plsc.subcore_barrier()

    wid = c * NS + s
    pltpu.sync_copy(packed_hbm.at[wid], packed_v)

    def _chunk(j, carry):
        for l in range(F // L):
            v = packed_v[j, pl.ds(l * L, L)]
            sring_v[0, pl.ds(l * L, L)] = v & MASK
            dring_v[0, pl.ds(l * L, L)] = lax.shift_right_logical(v, SHIFT)
        pltpu.async_copy(support_hbm.at[sring_v.at[0]], rows_v, sem).wait()
        pltpu.sync_copy(rows_v, acc_sh.at[dring_v.at[0]], add=True)
        return carry

    lax.fori_loop(0, jnp.where(c == 0, NCK0, NCK1), _chunk, 0)

    plsc.subcore_barrier()
    pltpu.sync_copy(acc_sh.at[pl.ds(base, ROWS_PER_TILE)],
                    out_hbm.at[c, pl.ds(base, ROWS_PER_TILE)])


_sc_scatter = functools.partial(
    pl.kernel,
    out_type=jax.ShapeDtypeStruct((NC, ACC_ROWS, F), jnp.float32),
    mesh=plsc.VectorSubcoreMesh(core_axis_name="c", subcore_axis_name="s"),
    scratch_types=[
        pltpu.VMEM((NCK_MAX, CH), jnp.int32),
        pltpu.VMEM((2, CH), jnp.int32),
        pltpu.VMEM((2, CH), jnp.int32),
        pltpu.VMEM((CH, F), jnp.float32),
        pltpu.VMEM_SHARED((ACC_ROWS, F), jnp.float32),
        pltpu.SemaphoreType.DMA,
    ],
)(_sc_scatter_kernel)


def kernel(h_v, edge_index, weight, bias):
    rows_blk = 1000
    support = pl.pallas_call(
        _matmul_body,
        grid=(N_NODES // rows_blk,),
        in_specs=[
            pl.BlockSpec((rows_blk, F), lambda i: (i, 0)),
            pl.BlockSpec((F, F), lambda i: (0, 0)),
        ],
        out_specs=pl.BlockSpec((rows_blk, F), lambda i: (i, 0)),
        out_shape=jax.ShapeDtypeStruct((N_NODES, F), jnp.float32),
    )(h_v, weight)

    ei = edge_index.astype(jnp.int32)
    flat = (jnp.pad(ei[0], (0, E_PAD - N_EDGES))
            | (jnp.pad(ei[1], (0, E_PAD - N_EDGES),
                       constant_values=DUMMY_DST) << SHIFT))

    fill = DUMMY_DST << SHIFT
    cut = NS * NCK0 * CH
    p0 = flat[:cut].reshape(NS, NCK0, CH)
    p1 = flat[cut:].reshape(NS, NCK1, CH)
    p0 = jnp.pad(p0, ((0, 0), (0, NCK_MAX - NCK0), (0, 0)),
                 constant_values=fill)
    p1 = jnp.pad(p1, ((0, 0), (0, NCK_MAX - NCK1), (0, 0)),
                 constant_values=fill)
    packed = jnp.concatenate([p0, p1], axis=0)

    partials = _sc_scatter(support, packed)

    out = pl.pallas_call(
        _combine_body,
        grid=(N_NODES // rows_blk,),
        in_specs=[
            pl.BlockSpec((rows_blk, F), lambda i: (i, 0)),
            pl.BlockSpec((rows_blk, F), lambda i: (i, 0)),
            pl.BlockSpec((1, F), lambda i: (0, 0)),
        ],
        out_specs=pl.BlockSpec((rows_blk, F), lambda i: (i, 0)),
        out_shape=jax.ShapeDtypeStruct((N_NODES, F), jnp.float32),
    )(partials[0, :N_NODES], partials[1, :N_NODES], bias.reshape(1, F))
    return out
